# baseline (speedup 1.0000x reference)
import jax
import jax.numpy as jnp
import numpy as np
from jax.experimental import pallas as pl
from jax.experimental.pallas import tpu as pltpu

_L = 2048
_D = 1024
_HQ = 16
_HKV = 4
_HD = 64
_NREP = _HQ // _HKV
_BLK = 128
_NB = _L // _BLK
_TOPK = 8
_QR = 256
_MQ = _NREP * _QR
_GW = _NREP * _HD
_SCALE = 1.0 / np.sqrt(_HD)
_NEG = -1e9


def _rope_packed(z, ct, st):
    lane = jax.lax.broadcasted_iota(jnp.int32, z.shape, 1)
    first = (lane & (_HD - 1)) < (_HD // 2)
    swapped = jnp.where(first, jnp.roll(z, -_HD // 2, axis=1),
                        jnp.roll(z, _HD // 2, axis=1))
    return z * ct + swapped * st


def _attn_body(x_ref, ct_ref, st_ref, wq_ref, wk_ref, wv_ref, wo_ref, e_ref,
               y_ref, kc_ref, vc_ref):
    qb = pl.program_id(0)
    g = pl.program_id(1)

    xb = x_ref[...]
    ct = ct_ref[...]
    st = st_ref[...]

    @pl.when(jnp.logical_and(qb == 0, g == 0))
    def _zero_v():
        vc_ref[...] = jnp.zeros((_HKV, _L, 2 * _HD), jnp.bfloat16)

    @pl.when(g == 0)
    def _kv():
        kp = jax.lax.dot_general(xb, wk_ref[...], (((1,), (0,)), ((), ())),
                                 preferred_element_type=jnp.float32)
        kr = _rope_packed(kp, ct, st)
        vp = jax.lax.dot_general(xb, wv_ref[...], (((1,), (0,)), ((), ())),
                                 preferred_element_type=jnp.float32)
        extra = (jax.lax.broadcasted_iota(jnp.int32, (_QR, _HD), 1) == 0
                 ).astype(jnp.float32)
        for gg in range(_HKV):
            kc_ref[gg, pl.ds(qb * _QR, _QR), :] = kr[:, gg * _HD:(gg + 1) * _HD]
            vc_ref[gg, pl.ds(qb * _QR, _QR), :] = jnp.concatenate(
                [vp[:, gg * _HD:(gg + 1) * _HD], extra],
                axis=1).astype(jnp.bfloat16)

    qp = jax.lax.dot_general(xb, wq_ref[g], (((1,), (0,)), ((), ())),
                             preferred_element_type=jnp.float32)
    qr = _rope_packed(qp, ct, st) * _SCALE
    qs = jnp.concatenate([qr[:, k * _HD:(k + 1) * _HD] for k in range(_NREP)],
                         axis=0)

    kk = kc_ref[g]
    scores = jax.lax.dot_general(qs, kk, (((1,), (1,)), ((), ())),
                                 preferred_element_type=jnp.float32)
    rr = jax.lax.broadcasted_iota(jnp.int32, (_MQ, _L), 0)
    row = qb * _QR + (rr & (_QR - 1))
    col = jax.lax.broadcasted_iota(jnp.int32, (_MQ, _L), 1)
    sc = jnp.where(col <= row, scores, _NEG)

    bms = [jnp.max(sc[:, j * _BLK:(j + 1) * _BLK], axis=1, keepdims=True)
           for j in range(_NB)]
    bm = jnp.concatenate(bms, axis=1)
    counts = jnp.zeros((_MQ, _NB), jnp.float32)
    for i in range(_NB):
        counts = counts + (bms[i] > bm).astype(jnp.float32)
    keepb = counts < float(_TOPK)
    m = jnp.max(jnp.where(keepb, bm, _NEG), axis=1, keepdims=True)

    keep_f = jax.lax.dot_general(keepb.astype(jnp.bfloat16), e_ref[...],
                                 (((1,), (0,)), ((), ())),
                                 preferred_element_type=jnp.float32)
    e = (jnp.exp(sc - m) * keep_f).astype(jnp.bfloat16)
    pv = jax.lax.dot_general(e, vc_ref[g], (((1,), (0,)), ((), ())),
                             preferred_element_type=jnp.float32)
    ov = pv[:, :_HD] / pv[:, _HD:_HD + 1]

    ovh = jnp.concatenate([ov[k * _QR:(k + 1) * _QR, :] for k in range(_NREP)],
                          axis=1)
    contrib = jax.lax.dot_general(ovh, wo_ref[g], (((1,), (0,)), ((), ())),
                                  preferred_element_type=jnp.float32)

    @pl.when(g == 0)
    def _init():
        y_ref[...] = contrib

    @pl.when(g > 0)
    def _acc():
        y_ref[...] = y_ref[...] + contrib


def kernel(x, freqs_cos, freqs_sin, wq, wk, wv, wo, start_pos):
    b, l, d = x.shape
    cos = jax.lax.dynamic_slice_in_dim(freqs_cos, start_pos, l, axis=0)
    sin = jax.lax.dynamic_slice_in_dim(freqs_sin, start_pos, l, axis=0)
    ctab = jnp.tile(jnp.concatenate([cos, cos], axis=1), (1, _NREP))
    stab = jnp.tile(jnp.concatenate([-sin, sin], axis=1), (1, _NREP))

    i = np.arange(_HD)
    src = np.where(i < _HD // 2, 2 * i, 2 * (i - _HD // 2) + 1)
    perm_q = (np.arange(_HQ)[:, None] * _HD + src[None, :]).reshape(-1)
    perm_k = (np.arange(_HKV)[:, None] * _HD + src[None, :]).reshape(-1)
    wq3 = jnp.transpose(wq[perm_q, :].reshape(_HKV, _GW, _D), (0, 2, 1))
    wk2 = jnp.transpose(wk[perm_k, :], (1, 0))
    wv2 = jnp.transpose(wv, (1, 0))
    wo3 = jnp.transpose(wo.reshape(_D, _HQ, _HD), (1, 2, 0)).reshape(
        _HKV, _GW, _D)
    expmat = (np.arange(_L)[None, :] // _BLK ==
              np.arange(_NB)[:, None]).astype(np.float32)
    expmat = jnp.asarray(expmat, dtype=jnp.bfloat16)
    x2 = x.reshape(l, d)

    y = pl.pallas_call(
        _attn_body,
        grid=(l // _QR, _HKV),
        in_specs=[
            pl.BlockSpec((_QR, _D), lambda qb, g: (qb, 0)),
            pl.BlockSpec((_QR, _GW), lambda qb, g: (qb, 0)),
            pl.BlockSpec((_QR, _GW), lambda qb, g: (qb, 0)),
            pl.BlockSpec((_HKV, _D, _GW), lambda qb, g: (0, 0, 0)),
            pl.BlockSpec((_D, _HKV * _HD), lambda qb, g: (0, 0)),
            pl.BlockSpec((_D, _HKV * _HD), lambda qb, g: (0, 0)),
            pl.BlockSpec((_HKV, _GW, _D), lambda qb, g: (0, 0, 0)),
            pl.BlockSpec((_NB, _L), lambda qb, g: (0, 0)),
        ],
        out_specs=pl.BlockSpec((_QR, _D), lambda qb, g: (qb, 0)),
        out_shape=jax.ShapeDtypeStruct((l, _D), jnp.float32),
        scratch_shapes=[
            pltpu.VMEM((_HKV, _L, _HD), jnp.float32),
            pltpu.VMEM((_HKV, _L, 2 * _HD), jnp.bfloat16),
        ],
        compiler_params=pltpu.CompilerParams(
            dimension_semantics=("arbitrary", "arbitrary")),
    )(x2, ctab, stab, wq3, wk2, wv2, wo3, expmat)
    return y.reshape(b, l, _D)


# speedup vs baseline: 1.0215x; 1.0215x over previous
import jax
import jax.numpy as jnp
import numpy as np
from jax.experimental import pallas as pl
from jax.experimental.pallas import tpu as pltpu

_L = 2048
_D = 1024
_HQ = 16
_HKV = 4
_HD = 64
_NREP = _HQ // _HKV
_BLK = 128
_NB = _L // _BLK
_TOPK = 8
_QR = 256
_MQ = _NREP * _QR
_GW = _NREP * _HD
_SCALE = 1.0 / np.sqrt(_HD)
_NEG = -1e9


def _rope_packed(z, ct, st):
    lane = jax.lax.broadcasted_iota(jnp.int32, z.shape, 1)
    first = (lane & (_HD - 1)) < (_HD // 2)
    swapped = jnp.where(first, jnp.roll(z, -_HD // 2, axis=1),
                        jnp.roll(z, _HD // 2, axis=1))
    return z * ct + swapped * st


def _attn_body(x_ref, ct_ref, st_ref, wq_ref, wk_ref, wv_ref, wo_ref, e_ref,
               cb_ref,
               y_ref, kc_ref, vc_ref):
    qb = pl.program_id(0)
    g = pl.program_id(1)

    xb = x_ref[...]
    ct = ct_ref[...]
    st = st_ref[...]

    @pl.when(jnp.logical_and(qb == 0, g == 0))
    def _zero_v():
        vc_ref[...] = jnp.zeros((_HKV, _L, 2 * _HD), jnp.float32)
        kc_ref[...] = jnp.zeros((_HKV, _L, _HD), jnp.float32)

    @pl.when(g == 0)
    def _kv():
        kp = jax.lax.dot_general(xb, wk_ref[...], (((1,), (0,)), ((), ())),
                                 preferred_element_type=jnp.float32)
        kr = _rope_packed(kp, ct, st)
        vp = jax.lax.dot_general(xb, wv_ref[...], (((1,), (0,)), ((), ())),
                                 preferred_element_type=jnp.float32)
        extra = (jax.lax.broadcasted_iota(jnp.int32, (_QR, _HD), 1) == 0
                 ).astype(jnp.float32)
        for gg in range(_HKV):
            kc_ref[gg, pl.ds(qb * _QR, _QR), :] = kr[:, gg * _HD:(gg + 1) * _HD]
            vc_ref[gg, pl.ds(qb * _QR, _QR), :] = jnp.concatenate(
                [vp[:, gg * _HD:(gg + 1) * _HD], extra], axis=1)

    qp = jax.lax.dot_general(xb, wq_ref[g], (((1,), (0,)), ((), ())),
                             preferred_element_type=jnp.float32)
    qr = _rope_packed(qp, ct, st) * _SCALE
    qs = jnp.concatenate([qr[:, k * _HD:(k + 1) * _HD] for k in range(_NREP)],
                         axis=0)

    kk = kc_ref[g]
    scores = jax.lax.dot_general(qs, kk, (((1,), (1,)), ((), ())),
                                 preferred_element_type=jnp.float32)
    cb = cb_ref[0]
    sb = scores + jnp.concatenate([cb] * _NREP, axis=0)

    bms = [jnp.max(sb[:, j * _BLK:(j + 1) * _BLK], axis=1, keepdims=True)
           for j in range(_NB)]
    bm = jnp.concatenate(bms, axis=1)
    counts = jnp.zeros((_MQ, _NB), jnp.float32)
    for i in range(_NB):
        counts = counts + (bms[i] > bm).astype(jnp.float32)
    keepb = counts < float(_TOPK)
    m = jnp.max(jnp.where(keepb, bm, _NEG), axis=1, keepdims=True)

    keep_f = jax.lax.dot_general(keepb.astype(jnp.float32), e_ref[...],
                                 (((1,), (0,)), ((), ())),
                                 preferred_element_type=jnp.float32)
    e = jnp.exp(sb - m) * keep_f
    pv = jax.lax.dot_general(e, vc_ref[g], (((1,), (0,)), ((), ())),
                             preferred_element_type=jnp.float32)
    ov = pv[:, :_HD] / pv[:, _HD:_HD + 1]

    ovh = jnp.concatenate([ov[k * _QR:(k + 1) * _QR, :] for k in range(_NREP)],
                          axis=1)
    contrib = jax.lax.dot_general(ovh, wo_ref[g], (((1,), (0,)), ((), ())),
                                  preferred_element_type=jnp.float32)

    @pl.when(g == 0)
    def _init():
        y_ref[...] = contrib

    @pl.when(g > 0)
    def _acc():
        y_ref[...] = y_ref[...] + contrib


def kernel(x, freqs_cos, freqs_sin, wq, wk, wv, wo, start_pos):
    b, l, d = x.shape
    cos = jax.lax.dynamic_slice_in_dim(freqs_cos, start_pos, l, axis=0)
    sin = jax.lax.dynamic_slice_in_dim(freqs_sin, start_pos, l, axis=0)
    ctab = jnp.tile(jnp.concatenate([cos, cos], axis=1), (1, _NREP))
    stab = jnp.tile(jnp.concatenate([-sin, sin], axis=1), (1, _NREP))

    i = np.arange(_HD)
    src = np.where(i < _HD // 2, 2 * i, 2 * (i - _HD // 2) + 1)
    perm_q = (np.arange(_HQ)[:, None] * _HD + src[None, :]).reshape(-1)
    perm_k = (np.arange(_HKV)[:, None] * _HD + src[None, :]).reshape(-1)
    wq3 = jnp.transpose(wq[perm_q, :].reshape(_HKV, _GW, _D), (0, 2, 1))
    wk2 = jnp.transpose(wk[perm_k, :], (1, 0))
    wv2 = jnp.transpose(wv, (1, 0))
    wo3 = jnp.transpose(wo.reshape(_D, _HQ, _HD), (1, 2, 0)).reshape(
        _HKV, _GW, _D)
    expmat = (np.arange(_L)[None, :] // _BLK ==
              np.arange(_NB)[:, None]).astype(np.float32)
    expmat = jnp.asarray(expmat)
    rr = np.arange(_QR)[None, :, None]
    cc = np.arange(_L)[None, None, :]
    qq = np.arange(l // _QR)[:, None, None]
    cbias = np.where(cc <= qq * _QR + rr, 0.0, _NEG).astype(np.float32)
    cbias = jnp.asarray(cbias)
    x2 = x.reshape(l, d)

    y = pl.pallas_call(
        _attn_body,
        grid=(l // _QR, _HKV),
        in_specs=[
            pl.BlockSpec((_QR, _D), lambda qb, g: (qb, 0)),
            pl.BlockSpec((_QR, _GW), lambda qb, g: (qb, 0)),
            pl.BlockSpec((_QR, _GW), lambda qb, g: (qb, 0)),
            pl.BlockSpec((_HKV, _D, _GW), lambda qb, g: (0, 0, 0)),
            pl.BlockSpec((_D, _HKV * _HD), lambda qb, g: (0, 0)),
            pl.BlockSpec((_D, _HKV * _HD), lambda qb, g: (0, 0)),
            pl.BlockSpec((_HKV, _GW, _D), lambda qb, g: (0, 0, 0)),
            pl.BlockSpec((_NB, _L), lambda qb, g: (0, 0)),
            pl.BlockSpec((1, _QR, _L), lambda qb, g: (qb, 0, 0)),
        ],
        out_specs=pl.BlockSpec((_QR, _D), lambda qb, g: (qb, 0)),
        out_shape=jax.ShapeDtypeStruct((l, _D), jnp.float32),
        scratch_shapes=[
            pltpu.VMEM((_HKV, _L, _HD), jnp.float32),
            pltpu.VMEM((_HKV, _L, 2 * _HD), jnp.float32),
        ],
        compiler_params=pltpu.CompilerParams(
            dimension_semantics=("arbitrary", "arbitrary")),
    )(x2, ctab, stab, wq3, wk2, wv2, wo3, expmat, cbias)
    return y.reshape(b, l, _D)
